# baseline (speedup 1.0000x reference)
import functools

import jax
import jax.numpy as jnp
from jax import lax
from jax.experimental import pallas as pl
from jax.experimental.pallas import tpu as pltpu
from jax.experimental.pallas import tpu_sc as plsc

B, S, H, K, A = 4096, 512, 128, 512, 512
BB = 256
CB = 512
NW = 32
BPW = B // NW
NCH = 4
CR = BPW // NCH


def _tables_body(emb_ref, wa_ref, ba_ref, wv_ref, bv_ref,
                 prob_ref, val_ref, embsq_ref):
    emb = emb_ref[...]
    logits = lax.dot_general(
        emb, wa_ref[...], (((1,), (1,)), ((), ())),
        precision=lax.Precision.HIGHEST,
        preferred_element_type=jnp.float32) + ba_ref[...]
    prob_ref[...] = jax.nn.softmax(logits, axis=-1)
    val_ref[...] = jnp.sum(emb * wv_ref[...], axis=1,
                           keepdims=True) + bv_ref[0, 0]
    embsq_ref[...] = jnp.sum(emb * emb, axis=1, keepdims=True)


def _mlp_body(in_ref, w1_ref, b1_ref, w2_ref, b2_ref, w3_ref, b3_ref,
              emb_ref, nhe_ref, x_ref, idx1_ref, idx2_ref):
    def lin(x, w_ref, b_ref):
        return lax.dot_general(
            x, w_ref[...], (((1,), (1,)), ((), ())),
            preferred_element_type=jnp.float32) + b_ref[...]

    x = jnp.maximum(lin(in_ref[...], w1_ref, b1_ref), 0.0)
    x = jnp.maximum(lin(x, w2_ref, b2_ref), 0.0)
    x = jnp.maximum(lin(x, w3_ref, b3_ref), 0.0)
    x_ref[...] = x
    s = lax.dot_general(
        x, emb_ref[...], (((1,), (1,)), ((), ())),
        preferred_element_type=jnp.float32) + nhe_ref[...]
    ks = lax.broadcasted_iota(jnp.int32, (BB, K), 1)
    m1 = jnp.max(s, axis=1, keepdims=True)
    idx1 = jnp.min(jnp.where(s == m1, ks, K), axis=1, keepdims=True)
    idx1_ref[...] = idx1
    s2 = jnp.where(ks == idx1, jnp.float32(-3.0e38), s)
    m2 = jnp.max(s2, axis=1, keepdims=True)
    idx2_ref[...] = jnp.min(jnp.where(s2 == m2, ks, K), axis=1,
                            keepdims=True)


def _exact_rowsum(sq):
    acc = sq[0:8, :]
    for t in range(1, 16):
        acc = acc + sq[8 * t:8 * (t + 1), :]
    w = 4
    while w >= 1:
        acc = acc[:w, :] + acc[w:2 * w, :]
        w //= 2
    return acc


def _refine_body(xt_ref, embt_ref, valrow_ref, idx1_ref, idx2_ref,
                 idxf_ref, val_ref):
    xt = xt_ref[...]
    idx1 = idx1_ref[0]
    idx2 = idx2_ref[0]
    ksc = lax.broadcasted_iota(jnp.int32, (K, CB), 0)

    def exact_d(idx):
        oh = jnp.where(ksc == idx, 1.0, 0.0)
        et = lax.dot_general(
            embt_ref[...], oh, (((1,), (0,)), ((), ())),
            precision=lax.Precision.HIGHEST,
            preferred_element_type=jnp.float32)
        return _exact_rowsum((xt - et) ** 2)

    d1 = exact_d(idx1)
    d2 = exact_d(idx2)
    idxf = jnp.where(d1 < d2, idx1,
                     jnp.where(d2 < d1, idx2, jnp.minimum(idx1, idx2)))
    idxf_ref[0] = idxf
    ohf = jnp.where(ksc == idxf, 1.0, 0.0)
    val_ref[0] = lax.dot_general(
        valrow_ref[...], ohf, (((1,), (0,)), ((), ())),
        precision=lax.Precision.HIGHEST,
        preferred_element_type=jnp.float32)


def _tc_stage(inputs, W1, b1, W2, b2, W3, b3, emb, Wa, ba, Wv, bv):
    prob_t, val_t, embsq = pl.pallas_call(
        _tables_body,
        out_shape=[
            jax.ShapeDtypeStruct((K, A), jnp.float32),
            jax.ShapeDtypeStruct((K, 1), jnp.float32),
            jax.ShapeDtypeStruct((K, 1), jnp.float32),
        ],
    )(emb, Wa, ba.reshape(1, A), Wv, bv.reshape(1, 1))

    neg_half_embsq = (-0.5) * embsq.reshape(1, K)

    full = lambda shape: pl.BlockSpec(shape, lambda i: (0, 0))
    x, idx1, idx2 = pl.pallas_call(
        _mlp_body,
        grid=(B // BB,),
        in_specs=[
            pl.BlockSpec((BB, S), lambda i: (i, 0)),
            full((H, S)), full((1, H)),
            full((H, H)), full((1, H)),
            full((H, H)), full((1, H)),
            full((K, H)), full((1, K)),
        ],
        out_specs=[pl.BlockSpec((BB, H), lambda i: (i, 0)),
                   pl.BlockSpec((BB, 1), lambda i: (i, 0)),
                   pl.BlockSpec((BB, 1), lambda i: (i, 0))],
        out_shape=[jax.ShapeDtypeStruct((B, H), jnp.float32),
                   jax.ShapeDtypeStruct((B, 1), jnp.int32),
                   jax.ShapeDtypeStruct((B, 1), jnp.int32)],
    )(inputs, W1, b1.reshape(1, H), W2, b2.reshape(1, H),
      W3, b3.reshape(1, H), emb, neg_half_embsq)

    nb = B // CB
    idxf, value = pl.pallas_call(
        _refine_body,
        grid=(nb,),
        in_specs=[
            pl.BlockSpec((H, CB), lambda i: (0, i)),
            full((H, K)), full((1, K)),
            pl.BlockSpec((1, 1, CB), lambda i: (i, 0, 0)),
            pl.BlockSpec((1, 1, CB), lambda i: (i, 0, 0)),
        ],
        out_specs=[pl.BlockSpec((1, 1, CB), lambda i: (i, 0, 0)),
                   pl.BlockSpec((1, 1, CB), lambda i: (i, 0, 0))],
        out_shape=[jax.ShapeDtypeStruct((nb, 1, CB), jnp.int32),
                   jax.ShapeDtypeStruct((nb, 1, CB), jnp.float32)],
    )(x.T, emb.T, val_t.reshape(1, K),
      idx1.reshape(nb, 1, CB), idx2.reshape(nb, 1, CB))

    return prob_t, idxf.reshape(B), value.reshape(B, 1)


def _gather_body(prob_hbm, idx_hbm, act_hbm, idx_v, rows_v, *sems):
    wid = lax.axis_index("s") * 2 + lax.axis_index("c")
    base = wid * BPW
    pltpu.sync_copy(idx_hbm.at[pl.ds(base, BPW)], idx_v)
    cps = [
        pltpu.async_copy(prob_hbm.at[idx_v.at[pl.ds(c * CR, CR)]],
                         rows_v.at[pl.ds(c * CR, CR)], sems[c])
        for c in range(NCH)
    ]
    for c in range(NCH):
        cps[c].wait()
        pltpu.sync_copy(rows_v.at[pl.ds(c * CR, CR)],
                        act_hbm.at[pl.ds(base + c * CR, CR)])


@functools.cache
def _gather_call():
    return functools.partial(
        pl.kernel,
        mesh=plsc.VectorSubcoreMesh(core_axis_name="c", subcore_axis_name="s"),
        out_type=jax.ShapeDtypeStruct((B, A), jnp.float32),
        scratch_types=[
            pltpu.VMEM((BPW,), jnp.int32),
            pltpu.VMEM((BPW, A), jnp.float32),
        ] + [pltpu.SemaphoreType.DMA] * NCH,
    )(_gather_body)


def kernel(inputs, W1, b1, W2, b2, W3, b3, emb, Wa, ba, Wv, bv):
    prob_t, idx, value = _tc_stage(
        inputs, W1, b1, W2, b2, W3, b3, emb, Wa, ba, Wv, bv)
    actions_prob = _gather_call()(prob_t, idx)
    return actions_prob, value
